# Initial kernel scaffold; baseline (speedup 1.0000x reference)
import jax
import jax.numpy as jnp
from jax.experimental import pallas as pl


def kernel(slices, coords, freq_algn, indices):
    raise NotImplementedError("write your pallas kernel here")


# speedup vs baseline: 3.7549x; 3.7549x over previous
import functools

import jax
import jax.numpy as jnp
from jax import lax
from jax.experimental import pallas as pl
from jax.experimental.pallas import tpu as pltpu
from jax.experimental.pallas import tpu_sc as plsc

_N_SLICES = 1024
_NUM_FREQS = 64
_NY = 24
_NX = 24
_COORD_DIM = 4
_B = 16384
_NYQUIST = 1000.0
_D = _NY * _NX
_NC, _NS = 2, 16
_NW = _NC * _NS
_BPW = _B // _NW
_CH = 64
_NCHUNK = _BPW // _CH
_LW = _COORD_DIM + 1
_LPW = _BPW * _LW


def _body(table_hbm, coords_hbm, freq_hbm, idx_hbm, out_hbm, lab_hbm,
          idx_v, rows_v, coords_v, freq_v, lab_v,
          in_sem0, in_sem1, out_sem0, out_sem1, lab_sem):
    wid = lax.axis_index("s") * _NC + lax.axis_index("c")
    base = wid * _BPW
    in_sems = (in_sem0, in_sem1)
    out_sems = (out_sem0, out_sem1)

    pltpu.sync_copy(idx_hbm.at[wid], idx_v)
    pltpu.sync_copy(coords_hbm, coords_v)
    pltpu.sync_copy(freq_hbm, freq_v)

    gh = [None] * _NCHUNK
    oh = [None] * _NCHUNK
    gh[0] = pltpu.async_copy(table_hbm.at[idx_v.at[0]], rows_v.at[0], in_sems[0])

    groups_per_chunk = _CH // 16
    for q in range(_BPW // 16):
        iv = idx_v[q // groups_per_chunk, pl.ds((q % groups_per_chunk) * 16, 16)]
        s4 = (iv >> 6) * _COORD_DIM
        f_idx = iv & (_NUM_FREQS - 1)
        lane = q * 16 * _LW + lax.broadcasted_iota(jnp.int32, (16,), 0) * _LW
        for c in range(_COORD_DIM):
            vals = plsc.load_gather(coords_v, [s4 + c])
            plsc.store_scatter(lab_v, [lane + c], vals)
        fv = plsc.load_gather(freq_v, [f_idx]) * (1.0 / _NYQUIST)
        plsc.store_scatter(lab_v, [lane + _COORD_DIM], fv)
    lab_cp = pltpu.async_copy(lab_v, lab_hbm.at[pl.ds(base * _LW, _LPW)], lab_sem)

    for g in range(1, _NCHUNK):
        if g >= 2:
            oh[g - 2].wait()
        gh[g] = pltpu.async_copy(
            table_hbm.at[idx_v.at[g]], rows_v.at[g % 2], in_sems[g % 2])
        gh[g - 1].wait()
        oh[g - 1] = pltpu.async_copy(
            rows_v.at[(g - 1) % 2],
            out_hbm.at[pl.ds(base + (g - 1) * _CH, _CH)],
            out_sems[(g - 1) % 2])
    gh[_NCHUNK - 1].wait()
    oh[_NCHUNK - 1] = pltpu.async_copy(
        rows_v.at[(_NCHUNK - 1) % 2],
        out_hbm.at[pl.ds(base + (_NCHUNK - 1) * _CH, _CH)],
        out_sems[(_NCHUNK - 1) % 2])
    oh[_NCHUNK - 2].wait()
    oh[_NCHUNK - 1].wait()
    lab_cp.wait()


_sc_call = functools.partial(
    pl.kernel,
    out_type=(jax.ShapeDtypeStruct((_B, _D), jnp.float32),
              jax.ShapeDtypeStruct((_B * _LW,), jnp.float32)),
    mesh=plsc.VectorSubcoreMesh(core_axis_name="c", subcore_axis_name="s"),
    scratch_types=[
        pltpu.VMEM((_NCHUNK, _CH), jnp.int32),
        pltpu.VMEM((2, _CH, _D), jnp.float32),
        pltpu.VMEM((_N_SLICES * _COORD_DIM,), jnp.float32),
        pltpu.VMEM((_NUM_FREQS,), jnp.float32),
        pltpu.VMEM((_LPW,), jnp.float32),
        pltpu.SemaphoreType.DMA,
        pltpu.SemaphoreType.DMA,
        pltpu.SemaphoreType.DMA,
        pltpu.SemaphoreType.DMA,
        pltpu.SemaphoreType.DMA,
    ],
    compiler_params=pltpu.CompilerParams(
        needs_layout_passes=False, use_tc_tiling_on_sc=False),
)(_body)


def kernel(slices, coords, freq_algn, indices):
    table = slices.reshape(_N_SLICES * _NUM_FREQS, _D)
    idx = indices.astype(jnp.int32).reshape(_NW, _NCHUNK, _CH)
    samples, labels = _sc_call(table, coords.reshape(-1), freq_algn, idx)
    return samples.reshape(_B, 1, _NY, _NX), labels.reshape(_B, _LW)


# speedup vs baseline: 8.0217x; 2.1363x over previous
import functools

import jax
import jax.numpy as jnp
from jax import lax
from jax.experimental import pallas as pl
from jax.experimental.pallas import tpu as pltpu
from jax.experimental.pallas import tpu_sc as plsc

_N_SLICES = 1024
_NUM_FREQS = 64
_NY = 24
_NX = 24
_COORD_DIM = 4
_B = 16384
_NYQUIST = 1000.0
_P = _NY * _NX
_NC, _NS = 2, 16
_NW = _NC * _NS
_PPW = _P // _NW
_BPW = _B // _NW
_LW = _COORD_DIM + 1
_NG = _B // 16


def _body(tab_hbm, coords_hbm, freq_hbm, idx_hbm, out_hbm, lab_hbm,
          idx_v, slab_v, row_v, pidx_v, coords_v, freq_v, lab_v,
          slab_sem, row_sem0, row_sem1, lab_sem):
    wid = lax.axis_index("s") * _NC + lax.axis_index("c")
    p0 = wid * _PPW
    row_sems = (row_sem0, row_sem1)

    pltpu.sync_copy(idx_hbm, idx_v)
    pltpu.sync_copy(coords_hbm, coords_v)
    pltpu.sync_copy(freq_hbm, freq_v)

    iota = lax.broadcasted_iota(jnp.int32, (16,), 0)
    fbase = [(q * 16 + iota) * _P for q in range(_NUM_FREQS // 16)]

    def start_slab(p):
        for q in range(_NUM_FREQS // 16):
            pidx_v[pl.ds(q * 16, 16)] = fbase[q] + p
        return pltpu.async_copy(tab_hbm.at[pidx_v], slab_v, slab_sem)

    slab_cp = start_slab(p0)

    base = wid * _BPW
    for q in range(_BPW // 16):
        raw = idx_v[pl.ds(base + q * 16, 16)]
        s = raw >> 6
        f = raw & (_NUM_FREQS - 1)
        for c in range(_COORD_DIM):
            lab_v[c, pl.ds(q * 16, 16)] = plsc.load_gather(
                coords_v, [s + c * _N_SLICES])
        lab_v[_COORD_DIM, pl.ds(q * 16, 16)] = (
            plsc.load_gather(freq_v, [f]) * (1.0 / _NYQUIST))
    lab_cps = [
        pltpu.async_copy(lab_v.at[c], lab_hbm.at[c, pl.ds(base, _BPW)], lab_sem)
        for c in range(_LW)
    ]

    def gather_row(row_ref):
        def grp(i, _):
            raw = idx_v[pl.ds(i * 16, 16)]
            row_ref[pl.ds(i * 16, 16)] = plsc.load_gather(
                slab_v, [raw & (_NUM_FREQS - 1), raw >> 6])
            return 0
        lax.fori_loop(0, _NG, grp, 0)

    row_cp = [None] * _PPW
    for j in range(_PPW):
        slab_cp.wait()
        if j >= 2:
            row_cp[j - 2].wait()
        gather_row(row_v.at[j % 2])
        if j + 1 < _PPW:
            slab_cp = start_slab(p0 + j + 1)
        row_cp[j] = pltpu.async_copy(
            row_v.at[j % 2], out_hbm.at[p0 + j], row_sems[j % 2])
    row_cp[_PPW - 2].wait()
    row_cp[_PPW - 1].wait()
    for cp in lab_cps:
        cp.wait()


_sc_call = functools.partial(
    pl.kernel,
    out_type=(jax.ShapeDtypeStruct((_P, _B), jnp.float32),
              jax.ShapeDtypeStruct((_LW, _B), jnp.float32)),
    mesh=plsc.VectorSubcoreMesh(core_axis_name="c", subcore_axis_name="s"),
    scratch_types=[
        pltpu.VMEM((_B,), jnp.int32),
        pltpu.VMEM((_NUM_FREQS, _N_SLICES), jnp.float32),
        pltpu.VMEM((2, _B), jnp.float32),
        pltpu.VMEM((_NUM_FREQS,), jnp.int32),
        pltpu.VMEM((_COORD_DIM * _N_SLICES,), jnp.float32),
        pltpu.VMEM((_NUM_FREQS,), jnp.float32),
        pltpu.VMEM((_LW, _BPW), jnp.float32),
        pltpu.SemaphoreType.DMA,
        pltpu.SemaphoreType.DMA,
        pltpu.SemaphoreType.DMA,
        pltpu.SemaphoreType.DMA,
    ],
    compiler_params=pltpu.CompilerParams(
        needs_layout_passes=False, use_tc_tiling_on_sc=False),
)(_body)


def kernel(slices, coords, freq_algn, indices):
    tab = jnp.transpose(slices, (1, 2, 3, 0)).reshape(_NUM_FREQS * _P,
                                                      _N_SLICES)
    coords_t = jnp.transpose(coords, (1, 0)).reshape(-1)
    out_phys, lab_phys = _sc_call(tab, coords_t, freq_algn,
                                  indices.astype(jnp.int32))
    samples = jnp.transpose(
        out_phys.reshape(1, _NY, _NX, _B), (3, 0, 1, 2))
    labels = jnp.transpose(lab_phys, (1, 0))
    return samples, labels


# speedup vs baseline: 8.6042x; 1.0726x over previous
import functools

import jax
import jax.numpy as jnp
from jax import lax
from jax.experimental import pallas as pl
from jax.experimental.pallas import tpu as pltpu
from jax.experimental.pallas import tpu_sc as plsc

_N_SLICES = 1024
_NUM_FREQS = 64
_NY = 24
_NX = 24
_COORD_DIM = 4
_B = 16384
_NYQUIST = 1000.0
_P = _NY * _NX
_NC, _NS = 2, 16
_NW = _NC * _NS
_PPW = _P // _NW
_BPW = _B // _NW
_NG = _B // 16


def _body(tab_hbm, coords_hbm, freq_hbm, idx_hbm, out_hbm, lab_hbm,
          idx_v, slab_v, row_v, coords_v, freq_v, lab_v,
          slab_sem, row_sem0, row_sem1, lab_sem):
    wid = lax.axis_index("s") * _NC + lax.axis_index("c")
    row_sems = (row_sem0, row_sem1)

    pltpu.sync_copy(idx_hbm, idx_v)
    pltpu.sync_copy(coords_hbm, coords_v)
    pltpu.sync_copy(freq_hbm, freq_v)

    def start_slab(p):
        y = p // _NX
        xt = (p % _NX) // 8
        xi = p % 8
        return pltpu.async_copy(
            tab_hbm.at[:, y, xt, :, xi, :], slab_v, slab_sem)

    p0 = wid * _PPW
    slab_cp = start_slab(p0)

    base = wid * _BPW
    for q in range(_BPW // 16):
        raw = idx_v[pl.ds(base + q * 16, 16)]
        f = raw & (_NUM_FREQS - 1)
        sb = raw >> 13
        sl = (raw >> 6) & 127
        blk, col = q // 8, (q % 8) * 16
        for c in range(_COORD_DIM):
            lab_v[blk, c, pl.ds(col, 16)] = plsc.load_gather(
                coords_v, [sb, jnp.full((16,), c, jnp.int32), sl])
        lab_v[blk, _COORD_DIM, pl.ds(col, 16)] = (
            plsc.load_gather(freq_v, [f]) * (1.0 / _NYQUIST))
    lab_cp = pltpu.async_copy(
        lab_v, lab_hbm.at[pl.ds(wid * (_BPW // 128), _BPW // 128)], lab_sem)

    def gather_row(row_ref):
        def grp(i, _):
            raw = idx_v[pl.ds(i * 16, 16)]
            vals = plsc.load_gather(
                slab_v,
                [raw & (_NUM_FREQS - 1), raw >> 13, (raw >> 6) & 127])
            row_ref[i >> 3, pl.ds((i & 7) * 16, 16)] = vals
            return 0
        lax.fori_loop(0, _NG, grp, 0)

    row_cp = [None] * _PPW
    for j in range(_PPW):
        p = p0 + j
        slab_cp.wait()
        if j >= 2:
            row_cp[j - 2].wait()
        gather_row(row_v.at[j % 2])
        if j + 1 < _PPW:
            slab_cp = start_slab(p + 1)
        row_cp[j] = pltpu.async_copy(
            row_v.at[j % 2], out_hbm.at[p // 8, :, p % 8, :],
            row_sems[j % 2])
    row_cp[_PPW - 2].wait()
    row_cp[_PPW - 1].wait()
    lab_cp.wait()


_sc_call = functools.partial(
    pl.kernel,
    out_type=(
        jax.ShapeDtypeStruct((_P // 8, _B // 128, 8, 128), jnp.float32),
        jax.ShapeDtypeStruct((_B // 128, 8, 128), jnp.float32),
    ),
    mesh=plsc.VectorSubcoreMesh(core_axis_name="c", subcore_axis_name="s"),
    scratch_types=[
        pltpu.VMEM((_B,), jnp.int32),
        pltpu.VMEM((_NUM_FREQS, 8, 128), jnp.float32),
        pltpu.VMEM((2, _B // 128, 128), jnp.float32),
        pltpu.VMEM((8, _COORD_DIM, 128), jnp.float32),
        pltpu.VMEM((_NUM_FREQS,), jnp.float32),
        pltpu.VMEM((_BPW // 128, 8, 128), jnp.float32),
        pltpu.SemaphoreType.DMA,
        pltpu.SemaphoreType.DMA,
        pltpu.SemaphoreType.DMA,
        pltpu.SemaphoreType.DMA,
    ],
    compiler_params=pltpu.CompilerParams(
        needs_layout_passes=False, use_tc_tiling_on_sc=False),
)(_body)


def kernel(slices, coords, freq_algn, indices):
    tab6 = (slices.transpose(1, 2, 3, 0)
            .reshape(_NUM_FREQS, _NY, _NX // 8, 8, _N_SLICES // 128, 128)
            .transpose(0, 1, 2, 4, 3, 5))
    coords3 = coords.transpose(1, 0).reshape(
        _COORD_DIM, _N_SLICES // 128, 128).transpose(1, 0, 2)
    out6, lab6 = _sc_call(tab6, coords3, freq_algn,
                          indices.astype(jnp.int32))
    samples = (out6.reshape(_NY, _NX // 8, _B // 128, 8, 128)
               .transpose(2, 4, 0, 1, 3)
               .reshape(_B, _NY, _NX)[:, None, :, :])
    labels = lab6.transpose(0, 2, 1).reshape(_B, 8)[:, :_COORD_DIM + 1]
    return samples, labels


# speedup vs baseline: 20.4201x; 2.3733x over previous
import functools

import jax
import jax.numpy as jnp
from jax import lax
from jax.experimental import pallas as pl
from jax.experimental.pallas import tpu as pltpu
from jax.experimental.pallas import tpu_sc as plsc

_N_SLICES = 1024
_NUM_FREQS = 64
_NY = 24
_NX = 24
_COORD_DIM = 4
_B = 16384
_NYQUIST = 1000.0
_P = _NY * _NX
_NC, _NS = 2, 16
_NW = _NC * _NS
_PPW = _P // _NW
_BPW = _B // _NW
_NG = _B // 16


def _body(tab_hbm, coords_hbm, freq_hbm, idx_hbm, out_hbm, lab_hbm,
          idx_v, slab_v, row_v, coords_v, freq_v, lab_v,
          slab_sem, row_sem0, row_sem1, lab_sem):
    wid = lax.axis_index("s") * _NC + lax.axis_index("c")
    row_sems = (row_sem0, row_sem1)

    pltpu.sync_copy(idx_hbm, idx_v)
    pltpu.sync_copy(coords_hbm, coords_v)
    pltpu.sync_copy(freq_hbm, freq_v)

    def start_slab(p):
        y = p // _NX
        xt = (p % _NX) // 8
        xi = p % 8
        return pltpu.async_copy(
            tab_hbm.at[:, y, xt, :, xi, :], slab_v, slab_sem)

    p0 = wid * _PPW
    slab_cp = start_slab(p0)

    base = wid * _BPW
    for q in range(_BPW // 16):
        raw = idx_v[pl.ds(base + q * 16, 16)]
        f = raw & (_NUM_FREQS - 1)
        sb = raw >> 13
        sl = (raw >> 6) & 127
        blk, col = q // 8, (q % 8) * 16
        for c in range(_COORD_DIM):
            lab_v[blk, c, pl.ds(col, 16)] = plsc.load_gather(
                coords_v, [sb, jnp.full((16,), c, jnp.int32), sl])
        lab_v[blk, _COORD_DIM, pl.ds(col, 16)] = (
            plsc.load_gather(freq_v, [f]) * (1.0 / _NYQUIST))
    lab_cp = pltpu.async_copy(
        lab_v, lab_hbm.at[pl.ds(wid * (_BPW // 128), _BPW // 128)], lab_sem)

    def gather_row(row_ref):
        @plsc.parallel_loop(0, _NG, 1, unroll=8)
        def _(i):
            raw = idx_v[pl.ds(i * 16, 16)]
            vals = plsc.load_gather(
                slab_v,
                [raw & (_NUM_FREQS - 1), raw >> 13, (raw >> 6) & 127])
            row_ref[i >> 3, pl.ds((i & 7) * 16, 16)] = vals

    row_cp = [None] * _PPW
    for j in range(_PPW):
        p = p0 + j
        slab_cp.wait()
        if j >= 2:
            row_cp[j - 2].wait()
        gather_row(row_v.at[j % 2])
        if j + 1 < _PPW:
            slab_cp = start_slab(p + 1)
        row_cp[j] = pltpu.async_copy(
            row_v.at[j % 2], out_hbm.at[p // 8, :, p % 8, :],
            row_sems[j % 2])
    row_cp[_PPW - 2].wait()
    row_cp[_PPW - 1].wait()
    lab_cp.wait()


_sc_call = functools.partial(
    pl.kernel,
    out_type=(
        jax.ShapeDtypeStruct((_P // 8, _B // 128, 8, 128), jnp.float32),
        jax.ShapeDtypeStruct((_B // 128, 8, 128), jnp.float32),
    ),
    mesh=plsc.VectorSubcoreMesh(core_axis_name="c", subcore_axis_name="s"),
    scratch_types=[
        pltpu.VMEM((_B,), jnp.int32),
        pltpu.VMEM((_NUM_FREQS, 8, 128), jnp.float32),
        pltpu.VMEM((2, _B // 128, 128), jnp.float32),
        pltpu.VMEM((8, _COORD_DIM, 128), jnp.float32),
        pltpu.VMEM((_NUM_FREQS,), jnp.float32),
        pltpu.VMEM((_BPW // 128, 8, 128), jnp.float32),
        pltpu.SemaphoreType.DMA,
        pltpu.SemaphoreType.DMA,
        pltpu.SemaphoreType.DMA,
        pltpu.SemaphoreType.DMA,
    ],
    compiler_params=pltpu.CompilerParams(
        needs_layout_passes=False, use_tc_tiling_on_sc=False),
)(_body)


def kernel(slices, coords, freq_algn, indices):
    tab6 = (slices.transpose(1, 2, 3, 0)
            .reshape(_NUM_FREQS, _NY, _NX // 8, 8, _N_SLICES // 128, 128)
            .transpose(0, 1, 2, 4, 3, 5))
    coords3 = coords.transpose(1, 0).reshape(
        _COORD_DIM, _N_SLICES // 128, 128).transpose(1, 0, 2)
    out6, lab6 = _sc_call(tab6, coords3, freq_algn,
                          indices.astype(jnp.int32))
    samples = (out6.reshape(_NY, _NX // 8, _B // 128, 8, 128)
               .transpose(2, 4, 0, 1, 3)
               .reshape(_B, _NY, _NX)[:, None, :, :])
    labels = lab6.transpose(0, 2, 1).reshape(_B, 8)[:, :_COORD_DIM + 1]
    return samples, labels
